# baseline (speedup 1.0000x reference)
import functools

import jax
import jax.numpy as jnp
from jax import lax
from jax.experimental import pallas as pl
from jax.experimental.pallas import tpu as pltpu
from jax.experimental.pallas import tpu_sc as plsc

R = 128
V = 100000
L = 16
NC, NS = 2, 16
NW = NC * NS
ROWS_PER_W = R // NW
NVEC = V // L
UNROLL = 10

LN2 = 0.6931471805599453


def _lane_max(vec):
    acc = vec[0]
    for i in range(1, L):
        acc = jnp.maximum(acc, vec[i])
    return acc


def _lane_sum(vec):
    acc = vec[0]
    for i in range(1, L):
        acc = acc + vec[i]
    return acc


def _row_logsoftmax(row_v):
    def mx_body(i, m):
        for u in range(UNROLL):
            m = jnp.maximum(m, row_v[pl.ds((i * UNROLL + u) * L, L)])
        return m

    m = lax.fori_loop(0, NVEC // UNROLL, mx_body,
                      jnp.full((L,), -jnp.inf, jnp.float32))

    def sm_body(i, acc):
        for u in range(UNROLL):
            acc = acc + jnp.exp(row_v[pl.ds((i * UNROLL + u) * L, L)] - m)
        return acc

    sv = lax.fori_loop(0, NVEC // UNROLL, sm_body, jnp.zeros((L,), jnp.float32))

    M = _lane_max(m)
    M_b = jnp.full((L,), M, jnp.float32)
    s_tot = _lane_sum(sv * jnp.exp(m - M_b))
    s_b = jnp.full((L,), s_tot, jnp.float32)

    bits = lax.bitcast_convert_type(s_b, jnp.int32)
    y = bits.astype(jnp.float32) * (LN2 / (1 << 23)) - 127.0 * LN2
    for _ in range(3):
        y = y + s_b * jnp.exp(-y) - 1.0
    lse = M_b + y

    def out_body(i, _):
        for u in range(UNROLL):
            sl = pl.ds((i * UNROLL + u) * L, L)
            row_v[sl] = row_v[sl] - lse
        return 0

    lax.fori_loop(0, NVEC // UNROLL, out_body, 0)


_mesh = plsc.VectorSubcoreMesh(core_axis_name="c", subcore_axis_name="s")


@functools.partial(
    pl.kernel,
    mesh=_mesh,
    out_type=jax.ShapeDtypeStruct((R, V), jnp.float32),
    scratch_types=[pltpu.VMEM((V,), jnp.float32)],
)
def _logsoftmax_sc(x_hbm, out_hbm, row_v):
    wid = lax.axis_index("s") * NC + lax.axis_index("c")
    for r in range(ROWS_PER_W):
        row = wid * ROWS_PER_W + r
        pltpu.sync_copy(x_hbm.at[row], row_v)
        _row_logsoftmax(row_v)
        pltpu.sync_copy(row_v, out_hbm.at[row])


def kernel(logits):
    return _logsoftmax_sc(logits)
